# baseline (speedup 1.0000x reference)
import jax
import jax.numpy as jnp
from jax import lax
from jax.experimental import pallas as pl
from jax.experimental.pallas import tpu as pltpu
from jax.experimental.pallas import tpu_sc as plsc

_D = 64
_SCALE = 8.0
_NW = 32
_SUBW = 128
_CH = 128
_NSTREAM = _CH // _SUBW
_NBUF = 8
_ROW_UNROLL = 8


def _emb_kernel(n_idx: int):
    n_per_w = n_idx // _NW
    n_chunk = n_per_w // _CH
    assert n_per_w % _CH == 0 and n_chunk % _NBUF == 0

    mesh = plsc.VectorSubcoreMesh(core_axis_name="core",
                                  subcore_axis_name="subcore")

    @jax.jit
    def run(idx_flat, table):
        @pl.kernel(
            out_type=jax.ShapeDtypeStruct((n_idx, _D), jnp.float32),
            mesh=mesh,
            compiler_params=pltpu.CompilerParams(use_tc_tiling_on_sc=False),
            scratch_types=(
                [pltpu.VMEM((n_per_w,), jnp.int32)]
                + [pltpu.VMEM((_CH, _D), jnp.float32) for _ in range(_NBUF)]
                + [pltpu.SemaphoreType.DMA for _ in range(2 * _NBUF)]
            ),
        )
        def kern(table_hbm, idx_hbm, out_hbm, idx_v, *rest):
            bufs = rest[:_NBUF]
            gsems = rest[_NBUF:2 * _NBUF]
            osems = rest[2 * _NBUF:]

            wid = lax.axis_index("subcore") * 2 + lax.axis_index("core")
            base = wid * n_per_w

            pltpu.sync_copy(idx_hbm.at[pl.ds(base, n_per_w)], idx_v)

            def issue_gather(ch, buf, sem):
                for j in range(_CH // 16):
                    iv = idx_v[pl.ds(ch * _CH + j * 16, 16)]
                    pltpu.async_copy(
                        table_hbm.at[iv],
                        buf.at[pl.ds(j * 16, 16), :],
                        sem,
                    )

            def drain_gather(buf, sem):
                for j in range(_CH // 16):
                    iv = idx_v[pl.ds(j * 16, 16)]
                    pltpu.make_async_copy(
                        table_hbm.at[iv],
                        buf.at[pl.ds(j * 16, 16), :],
                        sem,
                    ).wait()

            def scale(buf):
                @pl.loop(0, _CH, step=_ROW_UNROLL)
                def _(r):
                    for dr in range(_ROW_UNROLL):
                        for c in range(_D // 16):
                            slc = (pl.ds(r + dr, 1), pl.ds(c * 16, 16))
                            buf.at[*slc][...] = buf.at[*slc][...] * _SCALE

            def out_rows(ch):
                return out_hbm.at[pl.ds(base + ch * _CH, _CH), :]

            for p in range(_NBUF - 1):
                issue_gather(p, bufs[p], gsems[p])

            @pl.loop(0, n_chunk, step=_NBUF)
            def _(c):
                for p in range(_NBUF):
                    ch = c + p
                    drain_gather(bufs[p], gsems[p])
                    scale(bufs[p])
                    pltpu.async_copy(bufs[p], out_rows(ch), osems[p])

                    q = (p + _NBUF - 1) % _NBUF
                    nxt = ch + _NBUF - 1

                    @pl.when(nxt < n_chunk)
                    def _():
                        @pl.when(nxt >= _NBUF)
                        def _():
                            pltpu.make_async_copy(
                                bufs[q], out_rows(nxt), osems[q]
                            ).wait()

                        issue_gather(nxt, bufs[q], gsems[q])

            for p in range(_NBUF):
                pltpu.make_async_copy(
                    bufs[p], out_rows(p), osems[p]
                ).wait()

        return kern(table, idx_flat)

    return run


def kernel(x, table):
    b, s = x.shape
    out = _emb_kernel(b * s)(x.reshape(-1), table)
    return out.reshape(b, s, _D)


# speedup vs baseline: 1.2245x; 1.2245x over previous
import jax
import jax.numpy as jnp
from jax import lax
from jax.experimental import pallas as pl
from jax.experimental.pallas import tpu as pltpu
from jax.experimental.pallas import tpu_sc as plsc

_D = 64
_SCALE = 8.0
_NW = 32
_SUBW = 128
_CH = 128
_NSTREAM = _CH // _SUBW
_NBUF = 4
_ROW_UNROLL = 8


def _emb_kernel(n_idx: int):
    n_per_w = n_idx // _NW
    n_chunk = n_per_w // _CH
    assert n_per_w % _CH == 0 and n_chunk % _NBUF == 0

    mesh = plsc.VectorSubcoreMesh(core_axis_name="core",
                                  subcore_axis_name="subcore")

    @jax.jit
    def run(idx_flat, table):
        @pl.kernel(
            out_type=jax.ShapeDtypeStruct((n_idx, 2 * _D), jnp.float32),
            mesh=mesh,
            compiler_params=pltpu.CompilerParams(use_tc_tiling_on_sc=False),
            scratch_types=(
                [pltpu.VMEM((n_per_w,), jnp.int32)]
                + [pltpu.VMEM((_CH, 2 * _D), jnp.float32) for _ in range(_NBUF)]
                + [pltpu.SemaphoreType.DMA for _ in range(2 * _NBUF)]
            ),
        )
        def kern(table_hbm, idx_hbm, out_hbm, idx_v, *rest):
            bufs = rest[:_NBUF]
            gsems = rest[_NBUF:2 * _NBUF]
            osems = rest[2 * _NBUF:]

            wid = lax.axis_index("subcore") * 2 + lax.axis_index("core")
            base = wid * n_per_w

            pltpu.sync_copy(idx_hbm.at[pl.ds(base, n_per_w)], idx_v)

            def issue_gather(ch, buf, sem):
                for j in range(_CH // 16):
                    iv = idx_v[pl.ds(ch * _CH + j * 16, 16)]
                    pltpu.async_copy(
                        table_hbm.at[iv],
                        buf.at[pl.ds(j * 16, 16), :],
                        sem,
                    )

            def drain_gather(buf, sem):
                for j in range(_CH // 16):
                    iv = idx_v[pl.ds(j * 16, 16)]
                    pltpu.make_async_copy(
                        table_hbm.at[iv],
                        buf.at[pl.ds(j * 16, 16), :],
                        sem,
                    ).wait()

            def scale(buf):
                @pl.loop(0, _CH, step=_ROW_UNROLL)
                def _(r):
                    for dr in range(_ROW_UNROLL):
                        for c in range(_D // 16):
                            slc = (pl.ds(r + dr, 1), pl.ds(c * 16, 16))
                            buf.at[*slc][...] = buf.at[*slc][...] * _SCALE

            def out_rows(ch):
                return out_hbm.at[pl.ds(base + ch * _CH, _CH), :]

            for p in range(_NBUF - 1):
                issue_gather(p, bufs[p], gsems[p])

            @pl.loop(0, n_chunk, step=_NBUF)
            def _(c):
                for p in range(_NBUF):
                    ch = c + p
                    drain_gather(bufs[p], gsems[p])
                    scale(bufs[p])
                    pltpu.async_copy(bufs[p], out_rows(ch), osems[p])

                    q = (p + _NBUF - 1) % _NBUF
                    nxt = ch + _NBUF - 1

                    @pl.when(nxt < n_chunk)
                    def _():
                        @pl.when(nxt >= _NBUF)
                        def _():
                            pltpu.make_async_copy(
                                bufs[q], out_rows(nxt), osems[q]
                            ).wait()

                        issue_gather(nxt, bufs[q], gsems[q])

            for p in range(_NBUF):
                pltpu.make_async_copy(
                    bufs[p], out_rows(p), osems[p]
                ).wait()

        table_p = jnp.pad(table, ((0, 0), (0, _D)))
        return kern(table_p, idx_flat)

    return run


def kernel(x, table):
    b, s = x.shape
    out = _emb_kernel(b * s)(x.reshape(-1), table)
    return out[:, :_D].reshape(b, s, _D)
